# baseline (speedup 1.0000x reference)
import functools

import jax
import jax.numpy as jnp
from jax import lax
from jax.experimental import pallas as pl
from jax.experimental.pallas import tpu as pltpu
from jax.experimental.pallas import tpu_sc as plsc

DIM = 32
FDIM = 2 * DIM
LANES = 16
BLK = 128
KW = 4


@functools.cache
def _build(b, l, vocab):
    info = plsc.get_sparse_core_info()
    nc, ns = info.num_cores, info.num_subcores
    nw = nc * ns
    assert b == nw * BLK, (b, nw)

    n_full = vocab // BLK
    tail = vocab - n_full * BLK
    per_w = n_full // nw
    n_extra = n_full - per_w * nw
    n_rnd = per_w // KW
    krem = per_w - n_rnd * KW
    vpad = (n_full + (1 if tail else 0)) * BLK
    vrows = vpad // 2

    mesh = plsc.VectorSubcoreMesh(core_axis_name="c", subcore_axis_name="s")

    @functools.partial(
        pl.kernel,
        mesh=mesh,
        out_type=jax.ShapeDtypeStruct((vrows, BLK), jnp.float32),
        compiler_params=pltpu.CompilerParams(needs_layout_passes=False),
        scratch_types=[
            pltpu.VMEM((DIM, KW * BLK), jnp.float32),
            pltpu.VMEM((DIM, KW * BLK), jnp.float32),
            pltpu.VMEM((KW * BLK // 2, BLK), jnp.float32),
            pltpu.SemaphoreType.DMA,
        ],
    )
    def repack(emb_hbm, bias_hbm, tail_hbm, fused_hbm, be_v, bb_v, fb_v, sem):
        wid = lax.axis_index("s") * nc + lax.axis_index("c")
        c_base = wid * per_w
        iota = lax.iota(jnp.int32, LANES)

        def round_body(c, k):
            pltpu.sync_copy(emb_hbm.at[:, pl.ds(c * BLK, k * BLK)],
                            be_v.at[:, pl.ds(0, k * BLK)])
            pltpu.sync_copy(bias_hbm.at[:, pl.ds(c * BLK, k * BLK)],
                            bb_v.at[:, pl.ds(0, k * BLK)])

            def mg_body(mg, _):
                v16 = iota + mg * LANES
                r16 = lax.shift_right_logical(v16, 1)
                c16 = jnp.bitwise_and(v16, jnp.int32(1)) * FDIM
                sl = pl.ds(mg * LANES, LANES)
                for d in range(DIM):
                    plsc.store_scatter(fb_v, [r16, c16 + d], be_v[d, sl])
                    plsc.store_scatter(fb_v, [r16, c16 + (d + DIM)], bb_v[d, sl])
                return 0

            lax.fori_loop(0, k * BLK // LANES, mg_body, 0)
            pltpu.async_copy(fb_v.at[pl.ds(0, k * BLK // 2)],
                             fused_hbm.at[pl.ds(c * (BLK // 2), k * BLK // 2)],
                             sem).wait()

        def rnd_loop(i, _):
            round_body(c_base + i * KW, KW)
            return 0

        lax.fori_loop(0, n_rnd, rnd_loop, 0)
        if krem:
            round_body(c_base + n_rnd * KW, krem)

        @pl.when(wid < n_extra)
        def _():
            round_body(per_w * nw + wid, 1)

        if tail:
            @pl.when(wid == n_extra)
            def _():
                pltpu.sync_copy(tail_hbm,
                                fused_hbm.at[pl.ds(n_full * (BLK // 2), BLK // 2)])

    @functools.partial(
        pl.kernel,
        mesh=mesh,
        out_type=jax.ShapeDtypeStruct((l, DIM, b), jnp.float32),
        compiler_params=pltpu.CompilerParams(needs_layout_passes=False),
        scratch_types=[
            pltpu.VMEM((l, BLK), jnp.int32),
            pltpu.VMEM((l, BLK), jnp.int32),
            pltpu.VMEM((l, BLK), jnp.float32),
            pltpu.VMEM((BLK, BLK), jnp.float32),
            pltpu.VMEM((BLK, BLK), jnp.float32),
            pltpu.VMEM((DIM, BLK), jnp.float32),
            pltpu.SemaphoreType.DMA,
            pltpu.SemaphoreType.DMA,
            pltpu.SemaphoreType.DMA,
        ],
    )
    def lookup(idx_hbm, inp_hbm, fused_hbm, out_hbm,
               idx_v, ridx_v, inp_v, grow0_v, grow1_v, obuf_v,
               sem0, sem1, sem_out):
        wid = lax.axis_index("s") * nc + lax.axis_index("c")
        b0 = wid * BLK
        iota = lax.iota(jnp.int32, LANES)

        pltpu.sync_copy(idx_hbm.at[:, pl.ds(b0, BLK)], idx_v)
        pltpu.sync_copy(inp_hbm.at[:, pl.ds(b0, BLK)], inp_v)

        vec_per_blk = BLK // LANES

        def mask_body(q, _):
            j = q // vec_per_blk
            k = (q % vec_per_blk) * LANES
            v = inp_v[j, pl.ds(k, LANES)]
            isnan = v != v
            inp_v[j, pl.ds(k, LANES)] = jnp.where(isnan, jnp.float32(0.0), v)
            ii = idx_v[j, pl.ds(k, LANES)]
            ii = jnp.where(isnan, jnp.int32(0), ii)
            idx_v[j, pl.ds(k, LANES)] = ii
            ridx_v[j, pl.ds(k, LANES)] = lax.shift_right_logical(ii, 1)
            return 0

        lax.fori_loop(0, l * vec_per_blk, mask_body, 0)

        sems = (sem0, sem1)
        grows = (grow0_v, grow1_v)

        def fire(cl, buf):
            pltpu.async_copy(fused_hbm.at[ridx_v.at[cl]], grows[buf], sems[buf])

        def drain(cl, buf):
            pltpu.make_async_copy(fused_hbm.at[ridx_v.at[cl]],
                                  grows[buf], sems[buf]).wait()

        dn = lax.GatherDimensionNumbers(
            offset_dims=(), collapsed_slice_dims=(0,), start_index_map=(0,))

        def compute_and_store(cl, buf):
            g = grows[buf]

            def mg_body(mg, _):
                msl = pl.ds(mg * LANES, LANES)
                sv = inp_v[cl, msl]
                pv = jnp.bitwise_and(idx_v[cl, msl], jnp.int32(1))
                for i in range(LANES):
                    t = mg * LANES + i
                    i16 = jnp.full((LANES, 1), i, jnp.int32)
                    s = lax.gather(sv, i16, dn, (1,),
                                   mode=lax.GatherScatterMode.PROMISE_IN_BOUNDS)
                    hi = lax.gather(pv, i16, dn, (1,),
                                    mode=lax.GatherScatterMode.PROMISE_IN_BOUNDS
                                    ) > jnp.int32(0)
                    t16 = jnp.full((LANES,), t, jnp.int32)
                    for h in range(DIM // LANES):
                        e = jnp.where(hi,
                                      g[t, pl.ds(FDIM + h * LANES, LANES)],
                                      g[t, pl.ds(h * LANES, LANES)])
                        bi = jnp.where(hi,
                                       g[t, pl.ds(FDIM + DIM + h * LANES, LANES)],
                                       g[t, pl.ds(DIM + h * LANES, LANES)])
                        x = e * s + bi
                        ex = (jnp.exp(jnp.minimum(x, jnp.float32(0.0)))
                              - jnp.float32(1.0))
                        y = jnp.where(x > jnp.float32(0.0), x, ex)
                        plsc.store_scatter(obuf_v, [iota + h * LANES, t16], y)
                return 0

            lax.fori_loop(0, vec_per_blk, mg_body, 0)
            pltpu.async_copy(obuf_v,
                             out_hbm.at[cl, :, pl.ds(b0, BLK)], sem_out).wait()

        fire(0, 0)

        def chunk_body(j, _):
            fire(2 * j + 1, 1)
            drain(2 * j, 0)
            compute_and_store(2 * j, 0)

            @pl.when(2 * j + 2 < l)
            def _():
                fire(2 * j + 2, 0)

            drain(2 * j + 1, 1)
            compute_and_store(2 * j + 1, 1)
            return 0

        lax.fori_loop(0, l // 2, chunk_body, 0)
        if l % 2:
            drain(l - 1, 0)
            compute_and_store(l - 1, 0)

    def call(input_tokens, inputs, embeddings, bias):
        idx_t = jnp.transpose(input_tokens)
        inp_t = jnp.transpose(inputs, (1, 2, 0)).reshape(l, b)
        emb_t = jnp.transpose(embeddings)
        bias_t = jnp.transpose(bias)
        if tail:
            tail_blk = jnp.concatenate(
                [embeddings[n_full * BLK:, :], bias[n_full * BLK:, :]], axis=1)
            tail_blk = jnp.pad(tail_blk, ((0, BLK - tail), (0, 0)))
            tail_blk = tail_blk.reshape(BLK // 2, 2 * FDIM)
        else:
            tail_blk = jnp.zeros((BLK // 2, 2 * FDIM), jnp.float32)
        fused = repack(emb_t, bias_t, tail_blk)
        out = lookup(idx_t, inp_t, fused)
        return jnp.transpose(out, (2, 0, 1))

    return call


def kernel(input_tokens, inputs, embeddings, bias):
    b, l = input_tokens.shape
    vocab = embeddings.shape[0]
    return _build(b, l, vocab)(input_tokens, inputs, embeddings, bias)


# speedup vs baseline: 2.5827x; 2.5827x over previous
import functools

import jax
import jax.numpy as jnp
from jax import lax
from jax.experimental import pallas as pl
from jax.experimental.pallas import tpu as pltpu
from jax.experimental.pallas import tpu_sc as plsc

DIM = 32
LANES = 16
BLK = 128
TSTRIP = 4096
TBLOCK = 8192
KW = 16


@functools.cache
def _build(b, l, vocab):
    info = plsc.get_sparse_core_info()
    nc, ns = info.num_cores, info.num_subcores
    nw = nc * ns
    assert b == nw * BLK, (b, nw)

    n_full = vocab // BLK
    tail = vocab - n_full * BLK
    per_w = n_full // nw
    n_extra = n_full - per_w * nw
    n_blk = per_w // KW
    krem = per_w - n_blk * KW
    fused_n = TBLOCK * (n_full + (1 if tail else 0))

    mesh = plsc.VectorSubcoreMesh(core_axis_name="c", subcore_axis_name="s")

    @functools.partial(
        pl.kernel,
        mesh=mesh,
        out_type=jax.ShapeDtypeStruct((fused_n // TSTRIP, DIM, BLK), jnp.float32),
        scratch_types=[
            pltpu.VMEM((DIM, KW * BLK), jnp.float32),
            pltpu.SemaphoreType.DMA,
        ],
    )
    def repack(emb_hbm, bias_hbm, tail_hbm, fused_hbm, buf_v, sem):
        wid = lax.axis_index("s") * nc + lax.axis_index("c")
        c_base = wid * per_w
        fused_3d = fused_hbm

        def round_body(tbl, soff, c, k):
            pltpu.sync_copy(tbl.at[:, pl.ds(c * BLK, k * BLK)],
                            buf_v.at[:, pl.ds(0, k * BLK)])
            copies = []
            for j in range(k):
                copies.append(pltpu.async_copy(
                    buf_v.at[:, pl.ds(j * BLK, BLK)],
                    fused_3d.at[(c + j) * 2 + soff],
                    sem))
            for cp in copies:
                cp.wait()

        for tbl, soff in ((emb_hbm, 0), (bias_hbm, 1)):
            def blk_body(i, _):
                round_body(tbl, soff, c_base + i * KW, KW)
                return 0

            lax.fori_loop(0, n_blk, blk_body, 0)
            if krem:
                round_body(tbl, soff, c_base + n_blk * KW, krem)

        @pl.when(wid < n_extra)
        def _():
            ce = per_w * nw + wid
            for tbl, soff in ((emb_hbm, 0), (bias_hbm, 1)):
                round_body(tbl, soff, ce, 1)

        if tail:
            @pl.when(wid == n_extra)
            def _():
                for soff in (0, 1):
                    pltpu.sync_copy(tail_hbm.at[soff],
                                    fused_3d.at[n_full * 2 + soff])

    @functools.partial(
        pl.kernel,
        mesh=mesh,
        out_type=jax.ShapeDtypeStruct((l, DIM, b), jnp.float32),
        scratch_types=[
            pltpu.VMEM((l, BLK), jnp.int32),
            pltpu.VMEM((l, BLK), jnp.float32),
            pltpu.VMEM((TSTRIP,), jnp.int32),
            pltpu.VMEM((TSTRIP,), jnp.int32),
            pltpu.VMEM((TSTRIP,), jnp.float32),
            pltpu.VMEM((TSTRIP,), jnp.float32),
            pltpu.VMEM((TSTRIP,), jnp.float32),
            pltpu.VMEM((TSTRIP,), jnp.float32),
            pltpu.VMEM((DIM, BLK), jnp.float32),
            pltpu.SemaphoreType.DMA,
            pltpu.SemaphoreType.DMA,
            pltpu.SemaphoreType.DMA,
        ],
    )
    def lookup(idx_hbm, inp_hbm, fused_hbm, out_hbm,
               idx_v, inp_v, eidx0_v, eidx1_v, gemb0_v, gemb1_v,
               gbias0_v, gbias1_v, obuf_v, sem0, sem1, sem_out):
        wid = lax.axis_index("s") * nc + lax.axis_index("c")
        b0 = wid * BLK
        emb_flat = fused_hbm.at[0, 0]
        bias_flat = fused_hbm.at[1, 0]

        pltpu.sync_copy(idx_hbm.at[:, pl.ds(b0, BLK)], idx_v)
        pltpu.sync_copy(inp_hbm.at[:, pl.ds(b0, BLK)], inp_v)

        vec_per_blk = BLK // LANES

        def mask_body(q, _):
            j = q // vec_per_blk
            k = (q % vec_per_blk) * LANES
            v = inp_v[j, pl.ds(k, LANES)]
            isnan = v != v
            inp_v[j, pl.ds(k, LANES)] = jnp.where(isnan, jnp.float32(0.0), v)
            ii = idx_v[j, pl.ds(k, LANES)]
            idx_v[j, pl.ds(k, LANES)] = jnp.where(isnan, jnp.int32(0), ii)
            return 0

        lax.fori_loop(0, l * vec_per_blk, mask_body, 0)

        sems = (sem0, sem1)
        eidxs = (eidx0_v, eidx1_v)
        gembs = (gemb0_v, gemb1_v)
        gbiases = (gbias0_v, gbias1_v)

        def build_and_fire(cl, buf):
            def g_body(g, _):
                sl = pl.ds(g * LANES, LANES)
                t16 = idx_v[cl, sl]
                base = (lax.shift_left(lax.shift_right_logical(t16, 7), 13)
                        + jnp.bitwise_and(t16, jnp.int32(BLK - 1)))
                for d in range(DIM):
                    eidxs[buf][pl.ds(d * BLK + g * LANES, LANES)] = (
                        base + jnp.int32(d * BLK))
                return 0

            lax.fori_loop(0, vec_per_blk, g_body, 0)
            pltpu.async_copy(emb_flat.at[eidxs[buf]],
                             gembs[buf], sems[buf])
            pltpu.async_copy(bias_flat.at[eidxs[buf]],
                             gbiases[buf], sems[buf])

        def drain(buf):
            pltpu.make_async_copy(emb_flat.at[eidxs[buf]],
                                  gembs[buf], sems[buf]).wait()
            pltpu.make_async_copy(bias_flat.at[eidxs[buf]],
                                  gbiases[buf], sems[buf]).wait()

        def compute_and_store(cl, buf):
            def cg_body(g, _):
                sl = pl.ds(g * LANES, LANES)
                s = inp_v[cl, sl]
                for d in range(DIM):
                    fl = pl.ds(d * BLK + g * LANES, LANES)
                    x = gembs[buf][fl] * s + gbiases[buf][fl]
                    e = jnp.exp(jnp.minimum(x, jnp.float32(0.0))) - jnp.float32(1.0)
                    obuf_v[d, sl] = jnp.where(x > jnp.float32(0.0), x, e)
                return 0

            lax.fori_loop(0, vec_per_blk, cg_body, 0)
            pltpu.async_copy(obuf_v,
                             out_hbm.at[cl, :, pl.ds(b0, BLK)], sem_out).wait()

        build_and_fire(0, 0)
        n_pairs = l // 2

        def chunk_body(j, _):
            build_and_fire(2 * j + 1, 1)
            drain(0)
            compute_and_store(2 * j, 0)

            @pl.when(2 * j + 2 < l)
            def _():
                build_and_fire(2 * j + 2, 0)

            drain(1)
            compute_and_store(2 * j + 1, 1)
            return 0

        lax.fori_loop(0, n_pairs, chunk_body, 0)
        if l % 2:
            drain(0)
            compute_and_store(l - 1, 0)

    def call(input_tokens, inputs, embeddings, bias):
        idx_t = jnp.transpose(input_tokens)
        inp_t = jnp.transpose(inputs, (1, 2, 0)).reshape(l, b)
        emb_t = jnp.transpose(embeddings)
        bias_t = jnp.transpose(bias)
        n_full = vocab // BLK
        tail = vocab - n_full * BLK
        if tail:
            pad = ((0, 0), (0, BLK - tail))
            tail_blk = jnp.stack([
                jnp.pad(jnp.transpose(embeddings[n_full * BLK:, :]), pad),
                jnp.pad(jnp.transpose(bias[n_full * BLK:, :]), pad),
            ])
        else:
            tail_blk = jnp.zeros((2, DIM, BLK), jnp.float32)
        fused = repack(emb_t, bias_t, tail_blk)
        out = lookup(idx_t, inp_t, fused)
        return jnp.transpose(out, (2, 0, 1))

    return call


def kernel(input_tokens, inputs, embeddings, bias):
    b, l = input_tokens.shape
    vocab = embeddings.shape[0]
    return _build(b, l, vocab)(input_tokens, inputs, embeddings, bias)
